# baseline (speedup 1.0000x reference)
import functools

import jax
import jax.numpy as jnp
from jax import lax
from jax.experimental import pallas as pl
from jax.experimental.pallas import tpu as pltpu
from jax.experimental.pallas import tpu_sc as plsc

KSEL = 2
BM = 256
NTILES = 32
CH = 64
LANES = 16


def _router_body(nn, ee, gmax, flat_ref, rw_ref, s1_ref, s2_ref, w1x_ref,
                 w2x_ref, bexp_ref, nb_ref, aux_ref):
    flat = flat_ref[...]
    rw = rw_ref[...]
    logits = lax.dot_general(flat, rw, (((1,), (1,)), ((), ())),
                             preferred_element_type=jnp.float32)
    mx = jnp.max(logits, axis=1, keepdims=True)
    ex = jnp.exp(logits - mx)
    probs = ex / jnp.sum(ex, axis=1, keepdims=True)

    eidx = lax.broadcasted_iota(jnp.int32, (nn, ee), 1)
    m1 = jnp.max(probs, axis=1, keepdims=True)
    i1 = jnp.min(jnp.where(probs == m1, eidx, ee), axis=1, keepdims=True)
    oh1 = eidx == i1
    probs2 = jnp.where(oh1, -jnp.inf, probs)
    m2 = jnp.max(probs2, axis=1, keepdims=True)
    i2 = jnp.min(jnp.where(probs2 == m2, eidx, ee), axis=1, keepdims=True)
    oh2 = eidx == i2

    denom = m1 + m2
    w1 = m1 / denom
    w2 = m2 / denom

    pmean = jnp.sum(probs, axis=0, keepdims=True) * (1.0 / nn)
    cnt = jnp.sum(oh1.astype(jnp.float32) + oh2.astype(jnp.float32),
                  axis=0, keepdims=True)
    aux_ref[...] = ee * jnp.sum(cnt * (1.0 / nn) * pmean, axis=1,
                                keepdims=True)

    a1 = oh1.astype(jnp.int32)
    a2 = oh2.astype(jnp.int32)

    def csum_tokens(a):
        s = a
        k = 1
        while k < nn:
            sh = jnp.concatenate(
                [jnp.zeros((k, ee), jnp.int32), s[:nn - k, :]], axis=0)
            s = s + sh
            k *= 2
        return s

    c1 = csum_tokens(a1)
    c2 = csum_tokens(a2)
    cnt1 = c1[nn - 1:nn, :]
    ctot = cnt1 + c2[nn - 1:nn, :]
    blocks = (ctot + (BM - 1)) // BM

    cb = blocks
    k = 1
    while k < ee:
        cb = cb + jnp.concatenate(
            [jnp.zeros((1, k), jnp.int32), cb[:, :ee - k]], axis=1)
        k *= 2
    off_pad = BM * (cb - blocks)

    biota = lax.broadcasted_iota(jnp.int32, (1, gmax), 1)
    be = jnp.zeros((1, gmax), jnp.int32)
    for e in range(ee):
        be = be + (biota >= cb[:, e:e + 1]).astype(jnp.int32)
    bexp_ref[...] = jnp.minimum(be, ee - 1)
    nb_ref[...] = cb[:, ee - 1:ee]

    r1 = jnp.sum(jnp.where(oh1, c1, 0), axis=1, keepdims=True) - 1
    r2 = jnp.sum(jnp.where(oh2, c2, 0), axis=1, keepdims=True) - 1
    offp1 = jnp.sum(jnp.where(oh1, off_pad, 0), axis=1, keepdims=True)
    offp2 = jnp.sum(jnp.where(oh2, off_pad, 0), axis=1, keepdims=True)
    base2 = jnp.sum(jnp.where(oh2, cnt1, 0), axis=1, keepdims=True)
    s1_ref[...] = (offp1 + r1).reshape(1, nn)
    s2_ref[...] = (offp2 + base2 + r2).reshape(1, nn)
    w1x_ref[...] = jnp.broadcast_to(w1, (nn, LANES))
    w2x_ref[...] = jnp.broadcast_to(w2, (nn, LANES))


def _make_router(nn, cc, ee, gmax):
    body = functools.partial(_router_body, nn, ee, gmax)
    return pl.pallas_call(
        body,
        out_shape=(
            jax.ShapeDtypeStruct((1, nn), jnp.int32),
            jax.ShapeDtypeStruct((1, nn), jnp.int32),
            jax.ShapeDtypeStruct((nn, LANES), jnp.float32),
            jax.ShapeDtypeStruct((nn, LANES), jnp.float32),
            jax.ShapeDtypeStruct((1, gmax), jnp.int32),
            jax.ShapeDtypeStruct((1, 1), jnp.int32),
            jax.ShapeDtypeStruct((1, 1), jnp.float32),
        ),
    )


def _ffn_body(bexp_ref, nb_ref, x_ref, w1_ref, b1_ref, w2_ref, b2_ref, o_ref,
              w1s_ref, w2s_ref):
    b = pl.program_id(0)
    be = bexp_ref[b]
    prev = bexp_ref[jnp.maximum(b - 1, 0)]
    fresh = jnp.logical_or(b == 0, be != prev)

    @pl.when(jnp.logical_and(fresh, b < nb_ref[0]))
    def _():
        w1s_ref[...] = w1_ref[0].astype(jnp.bfloat16)
        w2s_ref[...] = w2_ref[0].astype(jnp.bfloat16)

    @pl.when(b < nb_ref[0])
    def _():
        xb = x_ref[...].astype(jnp.bfloat16)
        h = jnp.dot(xb, w1s_ref[...], preferred_element_type=jnp.float32)
        h = h + b1_ref[pl.ds(be, 1), :]
        g = 0.5 * h * (1.0 + lax.erf(h * 0.7071067811865476))
        o = jnp.dot(g.astype(jnp.bfloat16), w2s_ref[...],
                    preferred_element_type=jnp.float32)
        o_ref[...] = o + b2_ref[pl.ds(be, 1), :]


def _make_ffn(mpad, cc, ee, ff, gmax):
    grid_spec = pltpu.PrefetchScalarGridSpec(
        num_scalar_prefetch=2,
        grid=(gmax,),
        in_specs=[
            pl.BlockSpec((BM, cc), lambda b, bexp, nb: (b, 0)),
            pl.BlockSpec((1, cc, ff), lambda b, bexp, nb: (bexp[b], 0, 0)),
            pl.BlockSpec((ee, ff), lambda b, bexp, nb: (0, 0)),
            pl.BlockSpec((1, ff, cc), lambda b, bexp, nb: (bexp[b], 0, 0)),
            pl.BlockSpec((ee, cc), lambda b, bexp, nb: (0, 0)),
        ],
        out_specs=pl.BlockSpec((BM, cc), lambda b, bexp, nb: (b, 0)),
        scratch_shapes=[
            pltpu.VMEM((cc, ff), jnp.bfloat16),
            pltpu.VMEM((ff, cc), jnp.bfloat16),
        ],
    )
    return pl.pallas_call(
        _ffn_body,
        grid_spec=grid_spec,
        out_shape=jax.ShapeDtypeStruct((mpad, cc), jnp.float32),
        compiler_params=pltpu.CompilerParams(
            dimension_semantics=("arbitrary",)),
    )


def _make_sc_scatter(nn, cc, mpad, dtype):
    npt = nn // NTILES
    mesh = plsc.VectorSubcoreMesh(core_axis_name="c", subcore_axis_name="s")

    @functools.partial(
        pl.kernel,
        mesh=mesh,
        out_type=jax.ShapeDtypeStruct((mpad, cc), dtype),
        scratch_types=[
            pltpu.VMEM((npt,), jnp.int32),
            pltpu.VMEM((npt,), jnp.int32),
            pltpu.VMEM((npt, cc), dtype),
            pltpu.SemaphoreType.DMA,
            pltpu.SemaphoreType.DMA,
        ],
    )
    def k(x_hbm, s1_hbm, s2_hbm, xs_hbm, idx1_v, idx2_v, rows_v, sem1, sem2):
        wid = lax.axis_index("s") * 2 + lax.axis_index("c")
        base = wid * npt
        pltpu.sync_copy(s1_hbm.at[pl.ds(base, npt)], idx1_v)
        pltpu.sync_copy(s2_hbm.at[pl.ds(base, npt)], idx2_v)
        pltpu.sync_copy(x_hbm.at[pl.ds(base, npt)], rows_v)
        cp1 = pltpu.async_copy(rows_v, xs_hbm.at[idx1_v], sem1)
        cp2 = pltpu.async_copy(rows_v, xs_hbm.at[idx2_v], sem2)
        cp1.wait()
        cp2.wait()

    return k


def _make_sc_combine(nn, cc, mpad):
    npt = nn // NTILES
    nch = npt // CH
    mesh = plsc.VectorSubcoreMesh(core_axis_name="c", subcore_axis_name="s")

    @functools.partial(
        pl.kernel,
        mesh=mesh,
        out_type=jax.ShapeDtypeStruct((nn, cc), jnp.float32),
        scratch_types=[
            pltpu.VMEM((CH,), jnp.int32),
            pltpu.VMEM((CH,), jnp.int32),
            pltpu.VMEM((CH, cc), jnp.float32),
            pltpu.VMEM((CH, cc), jnp.float32),
            pltpu.VMEM((CH, LANES), jnp.float32),
            pltpu.VMEM((CH, LANES), jnp.float32),
            pltpu.SemaphoreType.DMA,
            pltpu.SemaphoreType.DMA,
        ],
    )
    def k(eo_hbm, s1_hbm, s2_hbm, w1x_hbm, w2x_hbm, out_hbm,
          idx1_v, idx2_v, b1_v, b2_v, wv1, wv2, sem1, sem2):
        wid = lax.axis_index("s") * 2 + lax.axis_index("c")
        base = wid * npt
        nvec = cc // LANES

        for ch in range(nch):
            cb = base + ch * CH
            pltpu.sync_copy(s1_hbm.at[pl.ds(cb, CH)], idx1_v)
            pltpu.sync_copy(s2_hbm.at[pl.ds(cb, CH)], idx2_v)
            cp1 = pltpu.async_copy(eo_hbm.at[idx1_v], b1_v, sem1)
            cp2 = pltpu.async_copy(eo_hbm.at[idx2_v], b2_v, sem2)
            pltpu.sync_copy(w1x_hbm.at[pl.ds(cb, CH)], wv1)
            pltpu.sync_copy(w2x_hbm.at[pl.ds(cb, CH)], wv2)
            cp1.wait()
            cp2.wait()

            def row(i, _):
                wa = wv1[i, :]
                wb = wv2[i, :]
                for j in range(nvec):
                    sl = pl.ds(j * LANES, LANES)
                    b1_v[i, sl] = b1_v[i, sl] * wa + b2_v[i, sl] * wb
                return 0

            lax.fori_loop(0, CH, row, 0)
            pltpu.sync_copy(b1_v, out_hbm.at[pl.ds(cb, CH)])

    return k


def kernel(x, router_w, W1, B1, W2, B2):
    bb, tt, cc = x.shape
    ee, _, ff = W1.shape
    nn = bb * tt
    gmax = (nn * KSEL) // BM + (ee - 1)
    gmax += (-gmax) % 8
    mpad = gmax * BM

    flat = x.reshape(nn, cc)
    s1, s2, w1x, w2x, bexp, nb, aux = _make_router(nn, cc, ee, gmax)(
        flat, router_w)
    s1f = s1.reshape(nn)
    s2f = s2.reshape(nn)

    xs = _make_sc_scatter(nn, cc, mpad, jnp.float32)(flat, s1f, s2f)
    eo = _make_ffn(mpad, cc, ee, ff, gmax)(
        bexp.reshape(gmax), nb.reshape(1), xs, W1, B1, W2, B2)
    out = _make_sc_combine(nn, cc, mpad)(eo, s1f, s2f, w1x, w2x)
    return out.reshape(bb, tt, cc), aux.reshape(())


# speedup vs baseline: 1.0457x; 1.0457x over previous
import functools

import jax
import jax.numpy as jnp
from jax import lax
from jax.experimental import pallas as pl
from jax.experimental.pallas import tpu as pltpu
from jax.experimental.pallas import tpu_sc as plsc

KSEL = 2
BM = 256
NTILES = 32
CH = 64
LANES = 16


NCH_R = 4


def _router_body(nn, ee, gmax, flat_ref, rw_ref, s1_ref, s2_ref, w1x_ref,
                 w2x_ref, bexp_ref, nb_ref, aux_ref, oh1_s, oh2_s, pm_s):
    i = pl.program_id(0)
    chr_ = nn // NCH_R

    @pl.when(i < NCH_R)
    def _chunk():
        flat = flat_ref[...]
        rw = rw_ref[...]
        logits = lax.dot_general(flat, rw, (((1,), (1,)), ((), ())),
                                 preferred_element_type=jnp.float32)
        mx = jnp.max(logits, axis=1, keepdims=True)
        ex = jnp.exp(logits - mx)
        probs = ex / jnp.sum(ex, axis=1, keepdims=True)

        eidx = lax.broadcasted_iota(jnp.int32, (chr_, ee), 1)
        m1 = jnp.max(probs, axis=1, keepdims=True)
        i1 = jnp.min(jnp.where(probs == m1, eidx, ee), axis=1, keepdims=True)
        oh1 = eidx == i1
        probs2 = jnp.where(oh1, -jnp.inf, probs)
        m2 = jnp.max(probs2, axis=1, keepdims=True)
        i2 = jnp.min(jnp.where(probs2 == m2, eidx, ee), axis=1, keepdims=True)
        oh2 = eidx == i2

        denom = m1 + m2
        w1x_ref[...] = jnp.broadcast_to(m1 / denom, (chr_, LANES))
        w2x_ref[...] = jnp.broadcast_to(m2 / denom, (chr_, LANES))
        oh1_s[pl.ds(i * chr_, chr_), :] = oh1.astype(jnp.int32)
        oh2_s[pl.ds(i * chr_, chr_), :] = oh2.astype(jnp.int32)
        psum = jnp.sum(probs, axis=0, keepdims=True)

        @pl.when(i == 0)
        def _():
            pm_s[...] = psum

        @pl.when(i > 0)
        def _():
            pm_s[...] = pm_s[...] + psum

    @pl.when(i == NCH_R)
    def _dispatch():
        oh1 = oh1_s[...] > 0
        oh2 = oh2_s[...] > 0
        a1 = oh1_s[...]
        a2 = oh2_s[...]

        def csum_tokens(a):
            s = a
            k = 1
            while k < nn:
                sh = jnp.concatenate(
                    [jnp.zeros((k, ee), jnp.int32), s[:nn - k, :]], axis=0)
                s = s + sh
                k *= 2
            return s

        c1 = csum_tokens(a1)
        c2 = csum_tokens(a2)
        cnt1 = c1[nn - 1:nn, :]
        ctot = cnt1 + c2[nn - 1:nn, :]

        aux_ref[...] = ee * jnp.sum(
            ctot.astype(jnp.float32) * (1.0 / nn) * pm_s[...] * (1.0 / nn),
            axis=1, keepdims=True)

        blocks = (ctot + (BM - 1)) // BM
        cb = blocks
        k = 1
        while k < ee:
            cb = cb + jnp.concatenate(
                [jnp.zeros((1, k), jnp.int32), cb[:, :ee - k]], axis=1)
            k *= 2
        off_pad = BM * (cb - blocks)

        biota = lax.broadcasted_iota(jnp.int32, (1, gmax), 1)
        be = jnp.zeros((1, gmax), jnp.int32)
        for e in range(ee):
            be = be + (biota >= cb[:, e:e + 1]).astype(jnp.int32)
        bexp_ref[...] = jnp.minimum(be, ee - 1)
        nb_ref[...] = cb[:, ee - 1:ee]

        r1 = jnp.sum(jnp.where(oh1, c1, 0), axis=1, keepdims=True) - 1
        r2 = jnp.sum(jnp.where(oh2, c2, 0), axis=1, keepdims=True) - 1
        offp1 = jnp.sum(jnp.where(oh1, off_pad, 0), axis=1, keepdims=True)
        offp2 = jnp.sum(jnp.where(oh2, off_pad, 0), axis=1, keepdims=True)
        base2 = jnp.sum(jnp.where(oh2, cnt1, 0), axis=1, keepdims=True)
        s1_ref[...] = (offp1 + r1).reshape(1, nn)
        s2_ref[...] = (offp2 + base2 + r2).reshape(1, nn)


def _make_router(nn, cc, ee, gmax):
    body = functools.partial(_router_body, nn, ee, gmax)
    chr_ = nn // NCH_R
    last = NCH_R - 1
    return pl.pallas_call(
        body,
        grid=(NCH_R + 1,),
        in_specs=[
            pl.BlockSpec((chr_, cc), lambda i: (jnp.minimum(i, last), 0)),
            pl.BlockSpec((ee, cc), lambda i: (0, 0)),
        ],
        out_specs=(
            pl.BlockSpec((1, nn), lambda i: (0, 0)),
            pl.BlockSpec((1, nn), lambda i: (0, 0)),
            pl.BlockSpec((chr_, LANES), lambda i: (jnp.minimum(i, last), 0)),
            pl.BlockSpec((chr_, LANES), lambda i: (jnp.minimum(i, last), 0)),
            pl.BlockSpec((1, gmax), lambda i: (0, 0)),
            pl.BlockSpec((1, 1), lambda i: (0, 0)),
            pl.BlockSpec((1, 1), lambda i: (0, 0)),
        ),
        out_shape=(
            jax.ShapeDtypeStruct((1, nn), jnp.int32),
            jax.ShapeDtypeStruct((1, nn), jnp.int32),
            jax.ShapeDtypeStruct((nn, LANES), jnp.float32),
            jax.ShapeDtypeStruct((nn, LANES), jnp.float32),
            jax.ShapeDtypeStruct((1, gmax), jnp.int32),
            jax.ShapeDtypeStruct((1, 1), jnp.int32),
            jax.ShapeDtypeStruct((1, 1), jnp.float32),
        ),
        scratch_shapes=[
            pltpu.VMEM((nn, ee), jnp.int32),
            pltpu.VMEM((nn, ee), jnp.int32),
            pltpu.VMEM((1, ee), jnp.float32),
        ],
        compiler_params=pltpu.CompilerParams(
            dimension_semantics=("arbitrary",)),
    )


def _ffn_body(bexp_ref, nb_ref, x_ref, w1_ref, b1_ref, w2_ref, b2_ref, o_ref):
    b = pl.program_id(0)

    @pl.when(b < nb_ref[0])
    def _():
        be = bexp_ref[b]
        xb = x_ref[...]
        h = jnp.dot(xb, w1_ref[0], preferred_element_type=jnp.float32)
        h = h + b1_ref[pl.ds(be, 1), :]
        g = 0.5 * h * (1.0 + lax.erf(h * 0.7071067811865476))
        o = jnp.dot(g, w2_ref[0], preferred_element_type=jnp.float32)
        o_ref[...] = o + b2_ref[pl.ds(be, 1), :]


def _make_ffn(mpad, cc, ee, ff, gmax):
    grid_spec = pltpu.PrefetchScalarGridSpec(
        num_scalar_prefetch=2,
        grid=(gmax,),
        in_specs=[
            pl.BlockSpec((BM, cc), lambda b, bexp, nb: (b, 0)),
            pl.BlockSpec((1, cc, ff), lambda b, bexp, nb: (bexp[b], 0, 0)),
            pl.BlockSpec((ee, ff), lambda b, bexp, nb: (0, 0)),
            pl.BlockSpec((1, ff, cc), lambda b, bexp, nb: (bexp[b], 0, 0)),
            pl.BlockSpec((ee, cc), lambda b, bexp, nb: (0, 0)),
        ],
        out_specs=pl.BlockSpec((BM, cc), lambda b, bexp, nb: (b, 0)),
    )
    return pl.pallas_call(
        _ffn_body,
        grid_spec=grid_spec,
        out_shape=jax.ShapeDtypeStruct((mpad, cc), jnp.float32),
        compiler_params=pltpu.CompilerParams(
            dimension_semantics=("arbitrary",)),
    )


def _make_sc_scatter(nn, cc, mpad, dtype):
    npt = nn // NTILES
    mesh = plsc.VectorSubcoreMesh(core_axis_name="c", subcore_axis_name="s")

    @functools.partial(
        pl.kernel,
        mesh=mesh,
        out_type=jax.ShapeDtypeStruct((mpad, cc), dtype),
        scratch_types=[
            pltpu.VMEM((npt,), jnp.int32),
            pltpu.VMEM((npt,), jnp.int32),
            pltpu.VMEM((npt, cc), dtype),
            pltpu.SemaphoreType.DMA,
            pltpu.SemaphoreType.DMA,
        ],
    )
    def k(x_hbm, s1_hbm, s2_hbm, xs_hbm, idx1_v, idx2_v, rows_v, sem1, sem2):
        wid = lax.axis_index("s") * 2 + lax.axis_index("c")
        base = wid * npt
        pltpu.sync_copy(s1_hbm.at[pl.ds(base, npt)], idx1_v)
        pltpu.sync_copy(s2_hbm.at[pl.ds(base, npt)], idx2_v)
        pltpu.sync_copy(x_hbm.at[pl.ds(base, npt)], rows_v)
        cp1 = pltpu.async_copy(rows_v, xs_hbm.at[idx1_v], sem1)
        cp2 = pltpu.async_copy(rows_v, xs_hbm.at[idx2_v], sem2)
        cp1.wait()
        cp2.wait()

    return k


def _make_sc_combine(nn, cc, mpad):
    npt = nn // NTILES
    nch = npt // CH
    mesh = plsc.VectorSubcoreMesh(core_axis_name="c", subcore_axis_name="s")

    @functools.partial(
        pl.kernel,
        mesh=mesh,
        out_type=jax.ShapeDtypeStruct((nn, cc), jnp.float32),
        scratch_types=[
            pltpu.VMEM((CH,), jnp.int32),
            pltpu.VMEM((CH,), jnp.int32),
            pltpu.VMEM((CH, cc), jnp.float32),
            pltpu.VMEM((CH, cc), jnp.float32),
            pltpu.VMEM((CH, LANES), jnp.float32),
            pltpu.VMEM((CH, LANES), jnp.float32),
            pltpu.SemaphoreType.DMA,
            pltpu.SemaphoreType.DMA,
        ],
    )
    def k(eo_hbm, s1_hbm, s2_hbm, w1x_hbm, w2x_hbm, out_hbm,
          idx1_v, idx2_v, b1_v, b2_v, wv1, wv2, sem1, sem2):
        wid = lax.axis_index("s") * 2 + lax.axis_index("c")
        base = wid * npt
        nvec = cc // LANES

        for ch in range(nch):
            cb = base + ch * CH
            pltpu.sync_copy(s1_hbm.at[pl.ds(cb, CH)], idx1_v)
            pltpu.sync_copy(s2_hbm.at[pl.ds(cb, CH)], idx2_v)
            cp1 = pltpu.async_copy(eo_hbm.at[idx1_v], b1_v, sem1)
            cp2 = pltpu.async_copy(eo_hbm.at[idx2_v], b2_v, sem2)
            pltpu.sync_copy(w1x_hbm.at[pl.ds(cb, CH)], wv1)
            pltpu.sync_copy(w2x_hbm.at[pl.ds(cb, CH)], wv2)
            cp1.wait()
            cp2.wait()

            def row(i, _):
                wa = wv1[i, :]
                wb = wv2[i, :]
                for j in range(nvec):
                    sl = pl.ds(j * LANES, LANES)
                    b1_v[i, sl] = b1_v[i, sl] * wa + b2_v[i, sl] * wb
                return 0

            lax.fori_loop(0, CH, row, 0)
            pltpu.sync_copy(b1_v, out_hbm.at[pl.ds(cb, CH)])

    return k


def kernel(x, router_w, W1, B1, W2, B2):
    bb, tt, cc = x.shape
    ee, _, ff = W1.shape
    nn = bb * tt
    gmax = (nn * KSEL) // BM + (ee - 1)
    gmax += (-gmax) % 8
    mpad = gmax * BM

    flat = x.reshape(nn, cc)
    s1, s2, w1x, w2x, bexp, nb, aux = _make_router(nn, cc, ee, gmax)(
        flat, router_w)
    s1f = s1.reshape(nn)
    s2f = s2.reshape(nn)

    xs = _make_sc_scatter(nn, cc, mpad, jnp.float32)(flat, s1f, s2f)
    eo = _make_ffn(mpad, cc, ee, ff, gmax)(
        bexp.reshape(gmax), nb.reshape(1), xs, W1, B1, W2, B2)
    out = _make_sc_combine(nn, cc, mpad)(eo, s1f, s2f, w1x, w2x)
    return out.reshape(bb, tt, cc), aux.reshape(())
